# baseline (speedup 1.0000x reference)
import jax
import jax.numpy as jnp
from jax.experimental import pallas as pl
from jax.experimental.pallas import tpu as pltpu

N = 10000
HID_PAD = 512
CLS_PAD = 128
BM_A = 1000
BM = 400
P1 = N // BM_A
P2 = P1 + N // BM
P3 = P2 + N // BM


def _gcn_kernel(x_ref, w1_ref, b1_ref, w2_ref, adj_ref, b2_ref,
                o_ref, s1_ref, s2_ref):
    j = pl.program_id(0)

    @pl.when(j < P1)
    def _phase_a():
        s1_ref[pl.ds((j % P1) * BM_A, BM_A), :] = jnp.dot(
            x_ref[...].astype(jnp.bfloat16), w1_ref[...],
            preferred_element_type=jnp.float32,
        ).astype(jnp.bfloat16)

    @pl.when((j >= P1) & (j < P2))
    def _phase_b():
        a = adj_ref[...].astype(jnp.bfloat16)
        acc = jnp.dot(a, s1_ref[...], preferred_element_type=jnp.float32)
        h = jnp.maximum(acc + b1_ref[...], 0.0).astype(jnp.bfloat16)
        s2_ref[pl.ds(((j - P1) % (N // BM)) * BM, BM), :] = jnp.dot(
            h, w2_ref[...], preferred_element_type=jnp.float32
        ).astype(jnp.bfloat16)

    @pl.when(j >= P2)
    def _phase_c():
        a = adj_ref[...].astype(jnp.bfloat16)
        acc = jnp.dot(a, s2_ref[...], preferred_element_type=jnp.float32)
        o_ref[...] = acc + b2_ref[...]


def _x_index(j):
    return (jnp.minimum(j, P1 - 1), 0)


def _adj_index(j):
    jb = jnp.where(j < P1, 0, jnp.where(j < P2, j - P1, j - P2))
    return (jb, 0)


def _out_index(j):
    return (jnp.maximum(j - P2, 0), 0)


@jax.jit
def _run(x, adj, W1, b1, W2, b2):
    in_feat = x.shape[1]
    hid = W1.shape[1]
    ncls = W2.shape[1]

    w1p = jnp.zeros((in_feat, HID_PAD), jnp.bfloat16).at[:, :hid].set(
        W1.astype(jnp.bfloat16))
    b1p = jnp.zeros((1, HID_PAD), jnp.float32).at[0, :hid].set(b1)
    w2p = jnp.zeros((HID_PAD, CLS_PAD), jnp.bfloat16).at[:hid, :ncls].set(
        W2.astype(jnp.bfloat16))
    b2p = jnp.zeros((1, CLS_PAD), jnp.float32).at[0, :ncls].set(b2)

    outp = pl.pallas_call(
        _gcn_kernel,
        grid=(P3,),
        in_specs=[
            pl.BlockSpec((BM_A, in_feat), _x_index),
            pl.BlockSpec((in_feat, HID_PAD), lambda j: (0, 0)),
            pl.BlockSpec((1, HID_PAD), lambda j: (0, 0)),
            pl.BlockSpec((HID_PAD, CLS_PAD), lambda j: (0, 0)),
            pl.BlockSpec((BM, N), _adj_index),
            pl.BlockSpec((1, CLS_PAD), lambda j: (0, 0)),
        ],
        out_specs=pl.BlockSpec((BM, CLS_PAD), _out_index),
        out_shape=jax.ShapeDtypeStruct((N, CLS_PAD), jnp.float32),
        scratch_shapes=[
            pltpu.VMEM((N, HID_PAD), jnp.bfloat16),
            pltpu.VMEM((N, CLS_PAD), jnp.bfloat16),
        ],
        compiler_params=pltpu.CompilerParams(
            vmem_limit_bytes=64 * 1024 * 1024),
    )(x, w1p, b1p, w2p, adj, b2p)

    return outp[:, :ncls]


def kernel(x, adj, W1, b1, W2, b2):
    return _run(x, adj, W1, b1, W2, b2)
